# baseline (speedup 1.0000x reference)
import functools

import jax
import jax.numpy as jnp
from jax import lax
from jax.experimental import pallas as pl
from jax.experimental.pallas import tpu as pltpu
from jax.experimental.pallas import tpu_sc as plsc

N = 10000
E = 320000
D = 128
G = 100
C = 16

NC = 2
NS = 16
NW = NC * NS
CH = 128
KC = 84
EP = NW * KC * CH
N2 = 16 * 632
NA = 10072

_mesh = plsc.VectorSubcoreMesh(core_axis_name="c", subcore_axis_name="s")


@functools.partial(
    pl.kernel,
    out_type=(jax.ShapeDtypeStruct((N2,), jnp.float32),
              jax.ShapeDtypeStruct((N2,), jnp.float32)),
    mesh=_mesh,
    scratch_types=[
        pltpu.VMEM((KC, CH), jnp.int32),
        pltpu.VMEM((KC, CH), jnp.float32),
        pltpu.VMEM((640,), jnp.float32),
        pltpu.VMEM_SHARED((N2,), jnp.float32),
    ],
)
def _deg(dstp, wp, out0, out1, dst_v, w_v, stage, acc):
    c = lax.axis_index("c")
    s = lax.axis_index("s")
    g = c * NS + s
    pltpu.sync_copy(dstp.at[g], dst_v)
    pltpu.sync_copy(wp.at[g], w_v)

    def zstore(t, _):
        stage[pl.ds(t * 16, 16)] = jnp.zeros((16,), jnp.float32)
        return _

    lax.fori_loop(0, 40, zstore, None)
    pltpu.sync_copy(stage.at[pl.ds(0, 632)], acc.at[pl.ds(s * 632, 632)])
    plsc.subcore_barrier()

    def body(k, _):
        pltpu.sync_copy(w_v.at[k], acc.at[dst_v.at[k]], add=True)
        return _

    lax.fori_loop(0, KC, body, None)
    plsc.subcore_barrier()
    sl = pl.ds(s * 632, 632)
    pltpu.sync_copy(acc.at[sl], stage.at[pl.ds(0, 632)])

    @pl.when(c == 0)
    def _():
        pltpu.sync_copy(stage.at[pl.ds(0, 632)], out0.at[sl])

    @pl.when(c == 1)
    def _():
        pltpu.sync_copy(stage.at[pl.ds(0, 632)], out1.at[sl])


@functools.partial(
    pl.kernel,
    out_type=(jax.ShapeDtypeStruct((N2, D), jnp.float32),
              jax.ShapeDtypeStruct((N2, D), jnp.float32)),
    mesh=_mesh,
    scratch_types=[
        pltpu.VMEM((2, CH), jnp.int32),
        pltpu.VMEM((4, CH), jnp.int32),
        pltpu.VMEM((3, CH), jnp.float32),
        pltpu.VMEM((3, CH, D), jnp.float32),
        pltpu.VMEM_SHARED((NA, D), jnp.float32),
        pltpu.SemaphoreType.DMA,
        pltpu.SemaphoreType.DMA,
        pltpu.SemaphoreType.DMA,
        pltpu.SemaphoreType.DMA,
        pltpu.SemaphoreType.DMA,
    ],
)
def _prop(u, srcp, dstp, wp, out0, out1, src_v, dst_v, w_v, rows, acc,
          esem, gsem, ss0, ss1, ss2):
    c = lax.axis_index("c")
    s = lax.axis_index("s")
    g = c * NS + s
    ssem = (ss0, ss1, ss2)
    pltpu.sync_copy(srcp.at[g, 0], src_v.at[0])
    pltpu.sync_copy(dstp.at[g, 0], dst_v.at[0])
    pltpu.sync_copy(wp.at[g, 0], w_v.at[0])
    pltpu.async_copy(srcp.at[g, 1], src_v.at[1], esem)
    pltpu.async_copy(dstp.at[g, 1], dst_v.at[1], esem)
    pltpu.async_copy(wp.at[g, 1], w_v.at[1], esem)
    pltpu.async_copy(u.at[src_v.at[0]], rows.at[0], gsem)

    def zrow(r, _):
        for j in range(8):
            rows[1, r, pl.ds(j * 16, 16)] = jnp.zeros((16,), jnp.float32)
        return _

    lax.fori_loop(0, CH, zrow, None)
    base = s * 632

    def stripe_copy(dst_is_acc):
        for lim, szs in ((15, (128, 128, 128, 128, 120)),
                         (16, (128, 128, 128, 128, 80))):
            @pl.when((s < lim) if lim == 15 else (s == 15))
            def _():
                off = 0
                for nr in szs:
                    pltpu.sync_copy(rows.at[1, pl.ds(0, nr)],
                                    acc.at[pl.ds(base + off, nr)])
                    off += nr

    stripe_copy(True)
    plsc.subcore_barrier()

    def scale(sw, b):
        def scale16(t, _):
            ws = w_v[sw, pl.ds(t * 16, 16)]
            for l in range(16):
                e = t * 16 + l
                we = ws[l]
                for jj in range(8):
                    sl = pl.ds(jj * 16, 16)
                    rows[b, e, sl] = rows[b, e, sl] * we
            return _

        lax.fori_loop(0, 8, scale16, None)

    def block(i, _):
        k0 = i * 12
        for q in range(12):
            k = k0 + q
            b = q % 3

            @pl.when(k >= 2)
            def _():
                pltpu.make_async_copy(rows.at[(b + 1) % 3],
                                      acc.at[dst_v.at[0]],
                                      ssem[(b + 1) % 3]).wait()

            pltpu.make_async_copy(u.at[src_v.at[0]], rows.at[b],
                                  gsem).wait()

            @pl.when(k + 1 < KC)
            def _():
                pltpu.make_async_copy(srcp.at[g, 0], src_v.at[0],
                                      esem).wait()
                pltpu.make_async_copy(dstp.at[g, 0], dst_v.at[0],
                                      esem).wait()
                pltpu.make_async_copy(wp.at[g, 0], w_v.at[0], esem).wait()

            @pl.when(k + 2 < KC)
            def _():
                pltpu.async_copy(srcp.at[g, k + 2], src_v.at[q % 2], esem)
                pltpu.async_copy(dstp.at[g, k + 2], dst_v.at[(q + 2) % 4],
                                 esem)
                pltpu.async_copy(wp.at[g, k + 2], w_v.at[(q + 2) % 3], esem)

            @pl.when(k + 1 < KC)
            def _():
                pltpu.async_copy(u.at[src_v.at[(q + 1) % 2]],
                                 rows.at[(b + 1) % 3], gsem)

            scale(b, b)
            pltpu.async_copy(rows.at[b], acc.at[dst_v.at[q % 4]],
                             ssem[b], add=True)
        return _

    lax.fori_loop(0, KC // 12, block, None)
    for d in (KC - 2, KC - 1):
        pltpu.make_async_copy(rows.at[d % 3], acc.at[dst_v.at[0]],
                              ssem[d % 3]).wait()
    plsc.subcore_barrier()
    outs = (out0, out1)
    for lim in (15, 16):
        @pl.when((s < lim) if lim == 15 else (s == 15))
        def _():
            szs = (128, 128, 128, 128, 120 if lim == 15 else 80)
            off = 0
            for nr in szs:
                ds_acc = pl.ds(base + off, nr)
                pltpu.sync_copy(acc.at[ds_acc], rows.at[0, pl.ds(0, nr)])

                @pl.when(c == 0)
                def _():
                    pltpu.sync_copy(rows.at[0, pl.ds(0, nr)],
                                    out0.at[ds_acc])

                @pl.when(c == 1)
                def _():
                    pltpu.sync_copy(rows.at[0, pl.ds(0, nr)],
                                    out1.at[ds_acc])
                off += nr


BN = 1000


def _mm1_body(x_ref, w1_ref, deg_ref, u1_ref, dinv_ref):
    deg = 1.0 + deg_ref[:, 0:1] + deg_ref[:, 1:2]
    dinv = lax.rsqrt(deg)
    dinv_ref[...] = dinv
    u1_ref[...] = dinv * jnp.dot(x_ref[...], w1_ref[...],
                                 preferred_element_type=jnp.float32)


def _mm2_body(sc0_ref, sc1_ref, u1_ref, dinv_ref, b1_ref, w2_ref,
              h1_ref, u2_ref):
    dinv = dinv_ref[...]
    pre = dinv * (sc0_ref[...] + sc1_ref[...] + u1_ref[...]) + b1_ref[...]
    h1 = jnp.maximum(pre, 0.0)
    h1_ref[...] = h1
    u2_ref[...] = dinv * jnp.dot(h1, w2_ref[...],
                                 preferred_element_type=jnp.float32)


def _fin_body(sc0_ref, sc1_ref, u2_ref, dinv_ref, b2_ref, h1_ref, batch_ref,
              wc_ref, bc_ref, out_ref, s1_acc, s2_acc, cnt_acc):
    i = pl.program_id(0)
    nsteps = pl.num_programs(0)
    dinv = dinv_ref[...]
    pre = dinv * (sc0_ref[...] + sc1_ref[...] + u2_ref[...]) + b2_ref[...]
    h2 = jnp.maximum(pre, 0.0)
    gids = lax.broadcasted_iota(jnp.int32, (BN, 128), 1)
    onehot = (batch_ref[...] == gids).astype(jnp.float32)
    dn = (((0,), (0,)), ((), ()))
    p1 = lax.dot_general(onehot, h1_ref[...], dn,
                         preferred_element_type=jnp.float32)
    p2 = lax.dot_general(onehot, h2, dn,
                         preferred_element_type=jnp.float32)
    ones_col = jnp.ones((BN, 1), jnp.float32)
    pc = lax.dot_general(onehot, ones_col, dn,
                         preferred_element_type=jnp.float32)

    @pl.when(i == 0)
    def _():
        s1_acc[...] = p1
        s2_acc[...] = p2
        cnt_acc[...] = pc

    @pl.when(i > 0)
    def _():
        s1_acc[...] = s1_acc[...] + p1
        s2_acc[...] = s2_acc[...] + p2
        cnt_acc[...] = cnt_acc[...] + pc

    @pl.when(i == nsteps - 1)
    def _():
        raw = (jnp.dot(s1_acc[...], wc_ref[0],
                       preferred_element_type=jnp.float32) +
               jnp.dot(s2_acc[...], wc_ref[1],
                       preferred_element_type=jnp.float32))
        denom = jnp.maximum(cnt_acc[...], 1.0)
        logits = raw / denom + bc_ref[...]
        out_ref[...] = logits[:G, :]


def kernel(x, edge_index, edge_weights, batch, W1, b1, W2, b2, Wc, bc):
    f32 = jnp.float32
    src = edge_index[0]
    dst = edge_index[1]
    pad = jnp.arange(EP - E, dtype=jnp.int32) % N
    srcp = jnp.concatenate([src, pad]).reshape(NW, KC, CH)
    dstp = jnp.concatenate([dst, pad]).reshape(NW, KC, CH)
    wp = jnp.concatenate([edge_weights,
                          jnp.zeros((EP - E,), f32)]).reshape(NW, KC, CH)
    deg0, deg1 = _deg(dstp, wp)
    degT = jnp.stack([deg0[:N], deg1[:N]], axis=1)

    grid = N // BN
    mm1 = pl.pallas_call(
        _mm1_body,
        grid=(grid,),
        in_specs=[
            pl.BlockSpec((BN, D), lambda i: (i, 0)),
            pl.BlockSpec((D, D), lambda i: (0, 0)),
            pl.BlockSpec((BN, 2), lambda i: (i, 0)),
        ],
        out_specs=[
            pl.BlockSpec((BN, D), lambda i: (i, 0)),
            pl.BlockSpec((BN, 1), lambda i: (i, 0)),
        ],
        out_shape=[
            jax.ShapeDtypeStruct((N, D), f32),
            jax.ShapeDtypeStruct((N, 1), f32),
        ],
    )
    u1, dinv = mm1(x, W1, degT)

    s1a, s1b = _prop(u1, srcp, dstp, wp)

    mm2 = pl.pallas_call(
        _mm2_body,
        grid=(grid,),
        in_specs=[
            pl.BlockSpec((BN, D), lambda i: (i, 0)),
            pl.BlockSpec((BN, D), lambda i: (i, 0)),
            pl.BlockSpec((BN, D), lambda i: (i, 0)),
            pl.BlockSpec((BN, 1), lambda i: (i, 0)),
            pl.BlockSpec((1, D), lambda i: (0, 0)),
            pl.BlockSpec((D, D), lambda i: (0, 0)),
        ],
        out_specs=[
            pl.BlockSpec((BN, D), lambda i: (i, 0)),
            pl.BlockSpec((BN, D), lambda i: (i, 0)),
        ],
        out_shape=[
            jax.ShapeDtypeStruct((N, D), f32),
            jax.ShapeDtypeStruct((N, D), f32),
        ],
    )
    h1, u2 = mm2(s1a, s1b, u1, dinv, b1.reshape(1, D), W2)

    s2a, s2b = _prop(u2, srcp, dstp, wp)

    fin = pl.pallas_call(
        _fin_body,
        grid=(grid,),
        in_specs=[
            pl.BlockSpec((BN, D), lambda i: (i, 0)),
            pl.BlockSpec((BN, D), lambda i: (i, 0)),
            pl.BlockSpec((BN, D), lambda i: (i, 0)),
            pl.BlockSpec((BN, 1), lambda i: (i, 0)),
            pl.BlockSpec((1, D), lambda i: (0, 0)),
            pl.BlockSpec((BN, D), lambda i: (i, 0)),
            pl.BlockSpec((BN, 1), lambda i: (i, 0)),
            pl.BlockSpec((2, D, C), lambda i: (0, 0, 0)),
            pl.BlockSpec((1, C), lambda i: (0, 0)),
        ],
        out_specs=pl.BlockSpec((G, C), lambda i: (0, 0)),
        out_shape=jax.ShapeDtypeStruct((G, C), f32),
        scratch_shapes=[
            pltpu.VMEM((128, 128), f32),
            pltpu.VMEM((128, 128), f32),
            pltpu.VMEM((128, 1), f32),
        ],
    )
    logits = fin(s2a, s2b, u2, dinv, b2.reshape(1, D), h1,
                 batch.reshape(N, 1), Wc.reshape(2, D, C), bc.reshape(1, C))
    return logits


# speedup vs baseline: 1.0647x; 1.0647x over previous
import functools

import jax
import jax.numpy as jnp
from jax import lax
from jax.experimental import pallas as pl
from jax.experimental.pallas import tpu as pltpu
from jax.experimental.pallas import tpu_sc as plsc

N = 10000
E = 320000
D = 128
G = 100
C = 16

NC = 2
NS = 16
NW = NC * NS
CH = 128
NG = 10
GC = 8
KC = NG * GC
EP = NW * KC * CH
N2 = 16 * 632

_mesh = plsc.VectorSubcoreMesh(core_axis_name="c", subcore_axis_name="s")


@functools.partial(
    pl.kernel,
    out_type=(jax.ShapeDtypeStruct((N2,), jnp.float32),
              jax.ShapeDtypeStruct((N2,), jnp.float32)),
    mesh=_mesh,
    scratch_types=[
        pltpu.VMEM((NG, GC, CH), jnp.int32),
        pltpu.VMEM((NG, GC, CH), jnp.float32),
        pltpu.VMEM((640,), jnp.float32),
        pltpu.VMEM_SHARED((N2,), jnp.float32),
    ],
)
def _deg(dstp, wp, out0, out1, dst_v, w_v, stage, acc):
    c = lax.axis_index("c")
    s = lax.axis_index("s")
    g = c * NS + s
    pltpu.sync_copy(dstp.at[g], dst_v)
    pltpu.sync_copy(wp.at[g], w_v)

    def zstore(t, _):
        stage[pl.ds(t * 16, 16)] = jnp.zeros((16,), jnp.float32)
        return _

    lax.fori_loop(0, 40, zstore, None)
    pltpu.sync_copy(stage.at[pl.ds(0, 632)], acc.at[pl.ds(s * 632, 632)])
    plsc.subcore_barrier()

    def body(k, _):
        gi = k // GC
        j = k - gi * GC
        pltpu.sync_copy(w_v.at[gi, j], acc.at[dst_v.at[gi, j]], add=True)
        return _

    lax.fori_loop(0, KC, body, None)
    plsc.subcore_barrier()
    sl = pl.ds(s * 632, 632)
    pltpu.sync_copy(acc.at[sl], stage.at[pl.ds(0, 632)])

    @pl.when(c == 0)
    def _():
        pltpu.sync_copy(stage.at[pl.ds(0, 632)], out0.at[sl])

    @pl.when(c == 1)
    def _():
        pltpu.sync_copy(stage.at[pl.ds(0, 632)], out1.at[sl])


@functools.partial(
    pl.kernel,
    out_type=(jax.ShapeDtypeStruct((N2, D), jnp.float32),
              jax.ShapeDtypeStruct((N2, D), jnp.float32)),
    mesh=_mesh,
    scratch_types=[
        pltpu.VMEM((2, GC, CH), jnp.int32),
        pltpu.VMEM((2, GC, CH), jnp.int32),
        pltpu.VMEM((2, GC, CH), jnp.float32),
        pltpu.VMEM((2, CH, D), jnp.float32),
        pltpu.VMEM_SHARED((N2, D), jnp.float32),
        pltpu.SemaphoreType.DMA,
        pltpu.SemaphoreType.DMA,
        pltpu.SemaphoreType.DMA,
        pltpu.SemaphoreType.DMA,
        pltpu.SemaphoreType.DMA,
    ],
)
def _prop(u, srcp, dstp, wp, out0, out1, src_v, dst_v, w_v, rows, acc,
          esem, gs0, gs1, ss0, ss1):
    c = lax.axis_index("c")
    s = lax.axis_index("s")
    g = c * NS + s
    gsem = (gs0, gs1)
    ssem = (ss0, ss1)
    pltpu.sync_copy(srcp.at[g, 0], src_v.at[0])
    pltpu.sync_copy(dstp.at[g, 0], dst_v.at[0])
    pltpu.sync_copy(wp.at[g, 0], w_v.at[0])
    pltpu.async_copy(u.at[src_v.at[0, 0]], rows.at[0], gsem[0])

    def zrow(r, _):
        for j in range(8):
            rows[1, r, pl.ds(j * 16, 16)] = jnp.zeros((16,), jnp.float32)
        return _

    lax.fori_loop(0, CH, zrow, None)
    base = s * 632
    for t in range(5):
        nr = 128 if t < 4 else 120
        pltpu.sync_copy(rows.at[1, pl.ds(0, nr)],
                        acc.at[pl.ds(base + t * 128, nr)])
    plsc.subcore_barrier()

    def scale(es, j, b):
        def scale16(t, _):
            ws = w_v[es, j, pl.ds(t * 16, 16)]
            for l in range(16):
                e = t * 16 + l
                we = ws[l]
                for jj in range(8):
                    sl = pl.ds(jj * 16, 16)
                    rows[b, e, sl] = rows[b, e, sl] * we
            return _

        lax.fori_loop(0, 8, scale16, None)

    def group(gi, _):
        es = gi & 1
        os = 1 - es
        for j in range(GC):
            k = gi * GC + j
            b = j % 2

            @pl.when(k >= 1)
            def _():
                pltpu.make_async_copy(
                    rows.at[1 - b], acc.at[dst_v.at[0, 0]],
                    ssem[1 - b]).wait()

            if j == 0:
                @pl.when(gi + 1 < NG)
                def _():
                    pltpu.async_copy(srcp.at[g, gi + 1], src_v.at[os], esem)
                    pltpu.async_copy(dstp.at[g, gi + 1], dst_v.at[os], esem)
                    pltpu.async_copy(wp.at[g, gi + 1], w_v.at[os], esem)

            if j < GC - 1:
                pltpu.async_copy(u.at[src_v.at[es, j + 1]], rows.at[1 - b],
                                 gsem[1 - b])
            else:
                @pl.when(gi + 1 < NG)
                def _():
                    pltpu.make_async_copy(srcp.at[g, 0], src_v.at[0],
                                          esem).wait()
                    pltpu.make_async_copy(dstp.at[g, 0], dst_v.at[0],
                                          esem).wait()
                    pltpu.make_async_copy(wp.at[g, 0], w_v.at[0],
                                          esem).wait()
                    pltpu.async_copy(u.at[src_v.at[os, 0]], rows.at[1 - b],
                                     gsem[1 - b])

            pltpu.make_async_copy(u.at[src_v.at[0, 0]], rows.at[b],
                                  gsem[b]).wait()
            scale(es, j, b)
            pltpu.async_copy(rows.at[b], acc.at[dst_v.at[es, j]],
                             ssem[b], add=True)
        return _

    lax.fori_loop(0, NG, group, None)
    lb = (KC - 1) % 2
    pltpu.make_async_copy(rows.at[lb], acc.at[dst_v.at[0, 0]],
                          ssem[lb]).wait()
    plsc.subcore_barrier()
    for t in range(5):
        nr = 128 if t < 4 else 120
        ds_acc = pl.ds(base + t * 128, nr)
        pltpu.sync_copy(acc.at[ds_acc], rows.at[0, pl.ds(0, nr)])

        @pl.when(c == 0)
        def _():
            pltpu.sync_copy(rows.at[0, pl.ds(0, nr)], out0.at[ds_acc])

        @pl.when(c == 1)
        def _():
            pltpu.sync_copy(rows.at[0, pl.ds(0, nr)], out1.at[ds_acc])


BN = 1000


def _mm1_body(x_ref, w1_ref, deg_ref, u1_ref, dinv_ref):
    deg = 1.0 + deg_ref[:, 0:1] + deg_ref[:, 1:2]
    dinv = lax.rsqrt(deg)
    dinv_ref[...] = dinv
    u1_ref[...] = dinv * jnp.dot(x_ref[...], w1_ref[...],
                                 preferred_element_type=jnp.float32)


def _mm2_body(sc0_ref, sc1_ref, u1_ref, dinv_ref, b1_ref, w2_ref,
              h1_ref, u2_ref):
    dinv = dinv_ref[...]
    pre = dinv * (sc0_ref[...] + sc1_ref[...] + u1_ref[...]) + b1_ref[...]
    h1 = jnp.maximum(pre, 0.0)
    h1_ref[...] = h1
    u2_ref[...] = dinv * jnp.dot(h1, w2_ref[...],
                                 preferred_element_type=jnp.float32)


def _fin_body(sc0_ref, sc1_ref, u2_ref, dinv_ref, b2_ref, h1_ref, batch_ref,
              wc_ref, bc_ref, out_ref, s1_acc, s2_acc, cnt_acc):
    i = pl.program_id(0)
    nsteps = pl.num_programs(0)
    dinv = dinv_ref[...]
    pre = dinv * (sc0_ref[...] + sc1_ref[...] + u2_ref[...]) + b2_ref[...]
    h2 = jnp.maximum(pre, 0.0)
    gids = lax.broadcasted_iota(jnp.int32, (BN, 128), 1)
    onehot = (batch_ref[...] == gids).astype(jnp.float32)
    dn = (((0,), (0,)), ((), ()))
    p1 = lax.dot_general(onehot, h1_ref[...], dn,
                         preferred_element_type=jnp.float32)
    p2 = lax.dot_general(onehot, h2, dn,
                         preferred_element_type=jnp.float32)
    ones_col = jnp.ones((BN, 1), jnp.float32)
    pc = lax.dot_general(onehot, ones_col, dn,
                         preferred_element_type=jnp.float32)

    @pl.when(i == 0)
    def _():
        s1_acc[...] = p1
        s2_acc[...] = p2
        cnt_acc[...] = pc

    @pl.when(i > 0)
    def _():
        s1_acc[...] = s1_acc[...] + p1
        s2_acc[...] = s2_acc[...] + p2
        cnt_acc[...] = cnt_acc[...] + pc

    @pl.when(i == nsteps - 1)
    def _():
        raw = (jnp.dot(s1_acc[...], wc_ref[0],
                       preferred_element_type=jnp.float32) +
               jnp.dot(s2_acc[...], wc_ref[1],
                       preferred_element_type=jnp.float32))
        denom = jnp.maximum(cnt_acc[...], 1.0)
        logits = raw / denom + bc_ref[...]
        out_ref[...] = logits[:G, :]


def kernel(x, edge_index, edge_weights, batch, W1, b1, W2, b2, Wc, bc):
    f32 = jnp.float32
    src = edge_index[0]
    dst = edge_index[1]
    pad = jnp.arange(EP - E, dtype=jnp.int32) % N
    srcp = jnp.concatenate([src, pad]).reshape(NW, NG, GC, CH)
    dstp = jnp.concatenate([dst, pad]).reshape(NW, NG, GC, CH)
    wp = jnp.concatenate([edge_weights,
                          jnp.zeros((EP - E,), f32)]).reshape(NW, NG, GC, CH)
    deg0, deg1 = _deg(dstp, wp)
    degT = jnp.stack([deg0[:N], deg1[:N]], axis=1)

    grid = N // BN
    mm1 = pl.pallas_call(
        _mm1_body,
        grid=(grid,),
        in_specs=[
            pl.BlockSpec((BN, D), lambda i: (i, 0)),
            pl.BlockSpec((D, D), lambda i: (0, 0)),
            pl.BlockSpec((BN, 2), lambda i: (i, 0)),
        ],
        out_specs=[
            pl.BlockSpec((BN, D), lambda i: (i, 0)),
            pl.BlockSpec((BN, 1), lambda i: (i, 0)),
        ],
        out_shape=[
            jax.ShapeDtypeStruct((N, D), f32),
            jax.ShapeDtypeStruct((N, 1), f32),
        ],
    )
    u1, dinv = mm1(x, W1, degT)

    s1a, s1b = _prop(u1, srcp, dstp, wp)

    mm2 = pl.pallas_call(
        _mm2_body,
        grid=(grid,),
        in_specs=[
            pl.BlockSpec((BN, D), lambda i: (i, 0)),
            pl.BlockSpec((BN, D), lambda i: (i, 0)),
            pl.BlockSpec((BN, D), lambda i: (i, 0)),
            pl.BlockSpec((BN, 1), lambda i: (i, 0)),
            pl.BlockSpec((1, D), lambda i: (0, 0)),
            pl.BlockSpec((D, D), lambda i: (0, 0)),
        ],
        out_specs=[
            pl.BlockSpec((BN, D), lambda i: (i, 0)),
            pl.BlockSpec((BN, D), lambda i: (i, 0)),
        ],
        out_shape=[
            jax.ShapeDtypeStruct((N, D), f32),
            jax.ShapeDtypeStruct((N, D), f32),
        ],
    )
    h1, u2 = mm2(s1a, s1b, u1, dinv, b1.reshape(1, D), W2)

    s2a, s2b = _prop(u2, srcp, dstp, wp)

    fin = pl.pallas_call(
        _fin_body,
        grid=(grid,),
        in_specs=[
            pl.BlockSpec((BN, D), lambda i: (i, 0)),
            pl.BlockSpec((BN, D), lambda i: (i, 0)),
            pl.BlockSpec((BN, D), lambda i: (i, 0)),
            pl.BlockSpec((BN, 1), lambda i: (i, 0)),
            pl.BlockSpec((1, D), lambda i: (0, 0)),
            pl.BlockSpec((BN, D), lambda i: (i, 0)),
            pl.BlockSpec((BN, 1), lambda i: (i, 0)),
            pl.BlockSpec((2, D, C), lambda i: (0, 0, 0)),
            pl.BlockSpec((1, C), lambda i: (0, 0)),
        ],
        out_specs=pl.BlockSpec((G, C), lambda i: (0, 0)),
        out_shape=jax.ShapeDtypeStruct((G, C), f32),
        scratch_shapes=[
            pltpu.VMEM((128, 128), f32),
            pltpu.VMEM((128, 128), f32),
            pltpu.VMEM((128, 1), f32),
        ],
    )
    logits = fin(s2a, s2b, u2, dinv, b2.reshape(1, D), h1,
                 batch.reshape(N, 1), Wc.reshape(2, D, C), bc.reshape(1, C))
    return logits


# speedup vs baseline: 1.0672x; 1.0023x over previous
import functools

import jax
import jax.numpy as jnp
from jax import lax
from jax.experimental import pallas as pl
from jax.experimental.pallas import tpu as pltpu
from jax.experimental.pallas import tpu_sc as plsc

N = 10000
E = 320000
D = 128
G = 100
C = 16

NC = 2
NS = 16
NW = NC * NS
CH = 128
NG = 10
GC = 8
KC = NG * GC
EP = NW * KC * CH
N2 = 16 * 632

_mesh = plsc.VectorSubcoreMesh(core_axis_name="c", subcore_axis_name="s",
                               num_cores=NC, num_subcores=NS)


@functools.partial(
    pl.kernel,
    out_type=(jax.ShapeDtypeStruct((N2,), jnp.float32),
              jax.ShapeDtypeStruct((N2,), jnp.float32)),
    mesh=_mesh,
    scratch_types=[
        pltpu.VMEM((NG, GC, CH), jnp.int32),
        pltpu.VMEM((NG, GC, CH), jnp.float32),
        pltpu.VMEM((640,), jnp.float32),
        pltpu.VMEM_SHARED((N2,), jnp.float32),
    ],
)
def _deg(dstp, wp, out0, out1, dst_v, w_v, stage, acc):
    c = lax.axis_index("c")
    s = lax.axis_index("s")
    g = c * NS + s
    pltpu.sync_copy(dstp.at[g], dst_v)
    pltpu.sync_copy(wp.at[g], w_v)

    def zstore(t, _):
        stage[pl.ds(t * 16, 16)] = jnp.zeros((16,), jnp.float32)
        return _

    lax.fori_loop(0, 40, zstore, None)
    pltpu.sync_copy(stage.at[pl.ds(0, 632)], acc.at[pl.ds(s * 632, 632)])
    plsc.subcore_barrier()

    def body(k, _):
        gi = k // GC
        j = k - gi * GC
        pltpu.sync_copy(w_v.at[gi, j], acc.at[dst_v.at[gi, j]], add=True)
        return _

    lax.fori_loop(0, KC, body, None)
    plsc.subcore_barrier()
    sl = pl.ds(s * 632, 632)
    pltpu.sync_copy(acc.at[sl], stage.at[pl.ds(0, 632)])

    @pl.when(c == 0)
    def _():
        pltpu.sync_copy(stage.at[pl.ds(0, 632)], out0.at[sl])

    @pl.when(c == 1)
    def _():
        pltpu.sync_copy(stage.at[pl.ds(0, 632)], out1.at[sl])


@functools.partial(
    pl.kernel,
    out_type=(jax.ShapeDtypeStruct((N2, D), jnp.float32),
              jax.ShapeDtypeStruct((N2, D), jnp.float32)),
    mesh=_mesh,
    scratch_types=[
        pltpu.VMEM((2, GC, CH), jnp.int32),
        pltpu.VMEM((2, GC, CH), jnp.int32),
        pltpu.VMEM((2, GC, CH), jnp.float32),
        pltpu.VMEM((2, CH, D), jnp.float32),
        pltpu.VMEM_SHARED((N2, D), jnp.float32),
        pltpu.SemaphoreType.DMA,
        pltpu.SemaphoreType.DMA,
        pltpu.SemaphoreType.DMA,
        pltpu.SemaphoreType.DMA,
        pltpu.SemaphoreType.DMA,
    ],
)
def _prop(u, srcp, dstp, wp, out0, out1, src_v, dst_v, w_v, rows, acc,
          esem, gs0, gs1, ss0, ss1):
    c = lax.axis_index("c")
    s = lax.axis_index("s")
    g = c * NS + s
    gsem = (gs0, gs1)
    ssem = (ss0, ss1)
    pltpu.sync_copy(srcp.at[g, 0], src_v.at[0])
    pltpu.sync_copy(dstp.at[g, 0], dst_v.at[0])
    pltpu.sync_copy(wp.at[g, 0], w_v.at[0])
    pltpu.async_copy(u.at[src_v.at[0, 0]], rows.at[0], gsem[0])

    def zrow(r, _):
        for j in range(8):
            rows[1, r, pl.ds(j * 16, 16)] = jnp.zeros((16,), jnp.float32)
        return _

    lax.fori_loop(0, CH, zrow, None)
    base = s * 632
    for t in range(5):
        nr = 128 if t < 4 else 120
        pltpu.sync_copy(rows.at[1, pl.ds(0, nr)],
                        acc.at[pl.ds(base + t * 128, nr)])
    plsc.subcore_barrier()

    def scale(es, j, b):
        def scale16(t, _):
            ws = w_v[es, j, pl.ds(t * 16, 16)]
            for l in range(16):
                e = t * 16 + l
                we = ws[l]
                for jj in range(8):
                    sl = pl.ds(jj * 16, 16)
                    rows[b, e, sl] = rows[b, e, sl] * we
            return _

        lax.fori_loop(0, 8, scale16, None)

    def group(gi, _):
        es = gi & 1
        os = 1 - es
        for j in range(GC):
            k = gi * GC + j
            b = j % 2

            @pl.when(k >= 1)
            def _():
                pltpu.make_async_copy(
                    rows.at[1 - b], acc.at[dst_v.at[0, 0]],
                    ssem[1 - b]).wait()

            if j == 0:
                @pl.when(gi + 1 < NG)
                def _():
                    pltpu.async_copy(srcp.at[g, gi + 1], src_v.at[os], esem)
                    pltpu.async_copy(dstp.at[g, gi + 1], dst_v.at[os], esem)
                    pltpu.async_copy(wp.at[g, gi + 1], w_v.at[os], esem)

            if j < GC - 1:
                pltpu.async_copy(u.at[src_v.at[es, j + 1]], rows.at[1 - b],
                                 gsem[1 - b])
            else:
                @pl.when(gi + 1 < NG)
                def _():
                    pltpu.make_async_copy(srcp.at[g, 0], src_v.at[0],
                                          esem).wait()
                    pltpu.make_async_copy(dstp.at[g, 0], dst_v.at[0],
                                          esem).wait()
                    pltpu.make_async_copy(wp.at[g, 0], w_v.at[0],
                                          esem).wait()
                    pltpu.async_copy(u.at[src_v.at[os, 0]], rows.at[1 - b],
                                     gsem[1 - b])

            pltpu.make_async_copy(u.at[src_v.at[0, 0]], rows.at[b],
                                  gsem[b]).wait()
            scale(es, j, b)
            pltpu.async_copy(rows.at[b], acc.at[dst_v.at[es, j]],
                             ssem[b], add=True)
        return _

    lax.fori_loop(0, NG, group, None)
    lb = (KC - 1) % 2
    pltpu.make_async_copy(rows.at[lb], acc.at[dst_v.at[0, 0]],
                          ssem[lb]).wait()
    plsc.subcore_barrier()
    for t in range(5):
        nr = 128 if t < 4 else 120
        ds_acc = pl.ds(base + t * 128, nr)
        pltpu.sync_copy(acc.at[ds_acc], rows.at[0, pl.ds(0, nr)])

        @pl.when(c == 0)
        def _():
            pltpu.sync_copy(rows.at[0, pl.ds(0, nr)], out0.at[ds_acc])

        @pl.when(c == 1)
        def _():
            pltpu.sync_copy(rows.at[0, pl.ds(0, nr)], out1.at[ds_acc])


BN = 1000


def _mm1_body(x_ref, w1_ref, deg_ref, u1_ref, dinv_ref):
    deg = 1.0 + deg_ref[:, 0:1] + deg_ref[:, 1:2]
    dinv = lax.rsqrt(deg)
    dinv_ref[...] = dinv
    u1_ref[...] = dinv * jnp.dot(x_ref[...], w1_ref[...],
                                 preferred_element_type=jnp.float32)


def _mm2_body(sc0_ref, sc1_ref, u1_ref, dinv_ref, b1_ref, w2_ref,
              h1_ref, u2_ref):
    dinv = dinv_ref[...]
    pre = dinv * (sc0_ref[...] + sc1_ref[...] + u1_ref[...]) + b1_ref[...]
    h1 = jnp.maximum(pre, 0.0)
    h1_ref[...] = h1
    u2_ref[...] = dinv * jnp.dot(h1, w2_ref[...],
                                 preferred_element_type=jnp.float32)


def _fin_body(sc0_ref, sc1_ref, u2_ref, dinv_ref, b2_ref, h1_ref, batch_ref,
              wc_ref, bc_ref, out_ref, s1_acc, s2_acc, cnt_acc):
    i = pl.program_id(0)
    nsteps = pl.num_programs(0)
    dinv = dinv_ref[...]
    pre = dinv * (sc0_ref[...] + sc1_ref[...] + u2_ref[...]) + b2_ref[...]
    h2 = jnp.maximum(pre, 0.0)
    gids = lax.broadcasted_iota(jnp.int32, (BN, 128), 1)
    onehot = (batch_ref[...] == gids).astype(jnp.float32)
    dn = (((0,), (0,)), ((), ()))
    p1 = lax.dot_general(onehot, h1_ref[...], dn,
                         preferred_element_type=jnp.float32)
    p2 = lax.dot_general(onehot, h2, dn,
                         preferred_element_type=jnp.float32)
    ones_col = jnp.ones((BN, 1), jnp.float32)
    pc = lax.dot_general(onehot, ones_col, dn,
                         preferred_element_type=jnp.float32)

    @pl.when(i == 0)
    def _():
        s1_acc[...] = p1
        s2_acc[...] = p2
        cnt_acc[...] = pc

    @pl.when(i > 0)
    def _():
        s1_acc[...] = s1_acc[...] + p1
        s2_acc[...] = s2_acc[...] + p2
        cnt_acc[...] = cnt_acc[...] + pc

    @pl.when(i == nsteps - 1)
    def _():
        raw = (jnp.dot(s1_acc[...], wc_ref[0],
                       preferred_element_type=jnp.float32) +
               jnp.dot(s2_acc[...], wc_ref[1],
                       preferred_element_type=jnp.float32))
        denom = jnp.maximum(cnt_acc[...], 1.0)
        logits = raw / denom + bc_ref[...]
        out_ref[...] = logits[:G, :]


def kernel(x, edge_index, edge_weights, batch, W1, b1, W2, b2, Wc, bc):
    f32 = jnp.float32
    src = edge_index[0]
    dst = edge_index[1]
    pad = jnp.arange(EP - E, dtype=jnp.int32) % N
    srcp = jnp.concatenate([src, pad]).reshape(NW, NG, GC, CH)
    dstp = jnp.concatenate([dst, pad]).reshape(NW, NG, GC, CH)
    wp = jnp.concatenate([edge_weights,
                          jnp.zeros((EP - E,), f32)]).reshape(NW, NG, GC, CH)
    deg0, deg1 = _deg(dstp, wp)
    degT = jnp.stack([deg0[:N], deg1[:N]], axis=1)

    grid = N // BN
    mm1 = pl.pallas_call(
        _mm1_body,
        grid=(grid,),
        in_specs=[
            pl.BlockSpec((BN, D), lambda i: (i, 0)),
            pl.BlockSpec((D, D), lambda i: (0, 0)),
            pl.BlockSpec((BN, 2), lambda i: (i, 0)),
        ],
        out_specs=[
            pl.BlockSpec((BN, D), lambda i: (i, 0)),
            pl.BlockSpec((BN, 1), lambda i: (i, 0)),
        ],
        out_shape=[
            jax.ShapeDtypeStruct((N, D), f32),
            jax.ShapeDtypeStruct((N, 1), f32),
        ],
    )
    u1, dinv = mm1(x, W1, degT)

    s1a, s1b = _prop(u1, srcp, dstp, wp)

    mm2 = pl.pallas_call(
        _mm2_body,
        grid=(grid,),
        in_specs=[
            pl.BlockSpec((BN, D), lambda i: (i, 0)),
            pl.BlockSpec((BN, D), lambda i: (i, 0)),
            pl.BlockSpec((BN, D), lambda i: (i, 0)),
            pl.BlockSpec((BN, 1), lambda i: (i, 0)),
            pl.BlockSpec((1, D), lambda i: (0, 0)),
            pl.BlockSpec((D, D), lambda i: (0, 0)),
        ],
        out_specs=[
            pl.BlockSpec((BN, D), lambda i: (i, 0)),
            pl.BlockSpec((BN, D), lambda i: (i, 0)),
        ],
        out_shape=[
            jax.ShapeDtypeStruct((N, D), f32),
            jax.ShapeDtypeStruct((N, D), f32),
        ],
    )
    h1, u2 = mm2(s1a, s1b, u1, dinv, b1.reshape(1, D), W2)

    s2a, s2b = _prop(u2, srcp, dstp, wp)

    fin = pl.pallas_call(
        _fin_body,
        grid=(grid,),
        in_specs=[
            pl.BlockSpec((BN, D), lambda i: (i, 0)),
            pl.BlockSpec((BN, D), lambda i: (i, 0)),
            pl.BlockSpec((BN, D), lambda i: (i, 0)),
            pl.BlockSpec((BN, 1), lambda i: (i, 0)),
            pl.BlockSpec((1, D), lambda i: (0, 0)),
            pl.BlockSpec((BN, D), lambda i: (i, 0)),
            pl.BlockSpec((BN, 1), lambda i: (i, 0)),
            pl.BlockSpec((2, D, C), lambda i: (0, 0, 0)),
            pl.BlockSpec((1, C), lambda i: (0, 0)),
        ],
        out_specs=pl.BlockSpec((G, C), lambda i: (0, 0)),
        out_shape=jax.ShapeDtypeStruct((G, C), f32),
        scratch_shapes=[
            pltpu.VMEM((128, 128), f32),
            pltpu.VMEM((128, 128), f32),
            pltpu.VMEM((128, 1), f32),
        ],
    )
    logits = fin(s2a, s2b, u2, dinv, b2.reshape(1, D), h1,
                 batch.reshape(N, 1), Wc.reshape(2, D, C), bc.reshape(1, C))
    return logits
